# baseline (speedup 1.0000x reference)
import functools

import jax
import jax.numpy as jnp
from jax import lax
from jax.experimental import pallas as pl
from jax.experimental.pallas import tpu as pltpu
from jax.experimental.pallas import tpu_sc as plsc

D = 64
B = 1024
T = 512
NC = 2
NS = 16
NW = NC * NS
B_PER_W = B // NW
NBUF = 2
LANES = 16


def _emb_kernel(idx_hbm, table_hbm, pos_hbm, out_hbm,
                idx_v, pos_v, rows0, rows1, gsem, osem):
    rows = (rows0, rows1)
    wid = lax.axis_index("s") * NC + lax.axis_index("c")
    b0 = wid * B_PER_W
    pltpu.sync_copy(pos_hbm.at[0], pos_v)
    pltpu.sync_copy(idx_hbm.at[pl.ds(b0, B_PER_W)], idx_v)

    def issue(i, j):
        pltpu.async_copy(table_hbm.at[idx_v.at[i]], rows[j], gsem.at[j])

    def wait_gather(i, j):
        pltpu.make_async_copy(
            table_hbm.at[idx_v.at[i]], rows[j], gsem.at[j]
        ).wait()

    def start_out(i, j):
        pltpu.async_copy(rows[j], out_hbm.at[b0 + i], osem.at[j])

    def wait_out(i, j):
        pltpu.make_async_copy(rows[j], out_hbm.at[b0 + i], osem.at[j]).wait()

    issue(0, 0)

    def group(g, carry):
        for j in range(NBUF):
            i = g * NBUF + j
            j2 = (j + 1) % NBUF

            @pl.when(i + 1 < B_PER_W)
            def _issue_ahead():
                @pl.when(i + 1 >= NBUF)
                def _wait_buf_free():
                    wait_out(i + 1 - NBUF, j2)

                issue(i + 1, j2)

            wait_gather(i, j)

            def row_body(r, c2):
                for c in range(D // LANES):
                    sl = pl.ds(c * LANES, LANES)
                    rows[j][r, sl] = rows[j][r, sl] + pos_v[r, sl]
                return c2

            lax.fori_loop(0, T, row_body, 0)
            start_out(i, j)
        return carry

    lax.fori_loop(0, B_PER_W // NBUF, group, 0)

    for j in range(NBUF):
        wait_out(B_PER_W - NBUF + j, j)


def kernel(inputs, table, pos):
    idx = inputs.astype(jnp.int32)
    posf = pos.astype(jnp.float32)

    mesh = plsc.VectorSubcoreMesh(core_axis_name="c", subcore_axis_name="s")
    run = functools.partial(
        pl.kernel,
        mesh=mesh,
        compiler_params=pltpu.CompilerParams(use_tc_tiling_on_sc=False),
        out_type=jax.ShapeDtypeStruct((B, T, D), jnp.float32),
        scratch_types=[
            pltpu.VMEM((B_PER_W, T), jnp.int32),
            pltpu.VMEM((T, D), jnp.float32),
            pltpu.VMEM((T, D), jnp.float32),
            pltpu.VMEM((T, D), jnp.float32),
            pltpu.SemaphoreType.DMA((NBUF,)),
            pltpu.SemaphoreType.DMA((NBUF,)),
        ],
    )(_emb_kernel)
    return run(idx, table, posf)
